# baseline (speedup 1.0000x reference)
import functools

import jax
import jax.numpy as jnp
from jax import lax
from jax.experimental import pallas as pl
from jax.experimental.pallas import tpu as pltpu
from jax.experimental.pallas import tpu_sc as plsc

C = 1000
CP1 = C + 1
CPAD = 1024
NSAMP = 1_000_000
BATCH = 4096
TEMPERATURE = 0.5
THRESHOLD = 0.95

NC, NS = 2, 16
NW = NC * NS
SCAT = 128
NSCAT = 248
CW = NSCAT * SCAT
NP = CW * NW
PADN = NP - NSAMP
PADROW = 1008
TAB = CPAD * CPAD
TPW = TAB // NS
ZCH = 1024


@functools.cache
def _sc_mesh():
    return plsc.VectorSubcoreMesh(core_axis_name="c", subcore_axis_name="s",
                                  num_cores=NC, num_subcores=NS)


def _hist_body(yh_hbm, yt_hbm, out_hbm, yh_v, yt_v, ones_v, zero_v, table,
               zsem, ssem, isem):
    cid = lax.axis_index("c")
    sid = lax.axis_index("s")
    wid = sid * NC + cid

    zeros16 = jnp.zeros((16,), jnp.float32)
    ones16 = jnp.ones((16,), jnp.float32)

    @pl.loop(0, ZCH // 16)
    def _fill_zero(i):
        zero_v[pl.ds(i * 16, 16)] = zeros16

    for k in range(SCAT // 16):
        ones_v[pl.ds(k * 16, 16)] = ones16

    QS = (64, 64, 64, 56)
    QO = (0, 64, 128, 192)

    def _load_q(q):
        base = wid * NSCAT + QO[q]
        return (pltpu.async_copy(yh_hbm.at[pl.ds(base, QS[q])],
                                 yh_v.at[pl.ds(QO[q], QS[q])], isem),
                pltpu.async_copy(yt_hbm.at[pl.ds(base, QS[q])],
                                 yt_v.at[pl.ds(QO[q], QS[q])], isem))

    inflight = _load_q(0)

    zcopies = [
        pltpu.async_copy(zero_v, table.at[pl.ds(sid * TPW + z * ZCH, ZCH)],
                         zsem)
        for z in range(TPW // ZCH)
    ]
    for zc in zcopies:
        zc.wait()
    plsc.subcore_barrier()

    GRP = 8
    for q in range(4):
        for ic in inflight:
            ic.wait()
        if q + 1 < 4:
            inflight = _load_q(q + 1)

        @pl.loop(0, QS[q] // GRP)
        def _scatter(g):
            copies = []
            for j2 in range(GRP):
                j = QO[q] + g * GRP + j2
                for k in range(SCAT // 16):
                    sl = pl.ds(k * 16, 16)
                    yh_v[j, sl] = yt_v[j, sl] * CPAD + yh_v[j, sl]
                copies.append(
                    pltpu.async_copy(ones_v, table.at[yh_v.at[j]], ssem,
                                     add=True))
            for cp in copies:
                cp.wait()

    plsc.subcore_barrier()
    pltpu.sync_copy(table.at[pl.ds(sid * TPW, TPW)], out_hbm.at[cid, sid])


@functools.cache
def _hist_sc():
    return pl.kernel(
        _hist_body,
        out_type=jax.ShapeDtypeStruct((NC, NS, TPW), jnp.float32),
        mesh=_sc_mesh(),
        scratch_types=[
            pltpu.VMEM((NSCAT, SCAT), jnp.int32),
            pltpu.VMEM((NSCAT, SCAT), jnp.int32),
            pltpu.VMEM((SCAT,), jnp.float32),
            pltpu.VMEM((ZCH,), jnp.float32),
            pltpu.VMEM_SHARED((TAB,), jnp.float32),
            pltpu.SemaphoreType.DMA,
            pltpu.SemaphoreType.DMA,
            pltpu.SemaphoreType.DMA,
        ],
    )


def _alpha_body(hp_ref, tmat_ref, py_ref, pyt_ref, hi_ref, mid_ref, lo_ref,
                beta_ref):
    h = hp_ref[0] + hp_ref[1]
    cnt = jnp.sum(h[:C, :], axis=0, keepdims=True)
    bmax = jnp.max(cnt)
    beta_ref[...] = (cnt / (2.0 * bmax - cnt)).T
    hT = h[:C, :C]
    un = h[:C, C:CP1]
    py = py_ref[...]
    tden = hT + un * py + 1.0
    tden = tden / jnp.sum(tden, axis=0, keepdims=True)
    tT = tmat_ref[...]
    alphaT = tT / tden
    err = (1.0 - THRESHOLD) * pyt_ref[...] / py
    am = jnp.where(jnp.abs(tT - tden) > err, alphaT, 1.0)
    amp = jnp.concatenate([am, jnp.zeros((C, CPAD - C), jnp.float32)], axis=1)
    hi = amp.astype(jnp.bfloat16)
    r1 = amp - hi.astype(jnp.float32)
    mid = r1.astype(jnp.bfloat16)
    lo = (r1 - mid.astype(jnp.float32)).astype(jnp.bfloat16)
    hi_ref[...] = hi
    mid_ref[...] = mid
    lo_ref[...] = lo


def _alpha_tc(hp, tmatT, py_row, pyt_col):
    return pl.pallas_call(
        _alpha_body,
        out_shape=[
            jax.ShapeDtypeStruct((C, CPAD), jnp.bfloat16),
            jax.ShapeDtypeStruct((C, CPAD), jnp.bfloat16),
            jax.ShapeDtypeStruct((C, CPAD), jnp.bfloat16),
            jax.ShapeDtypeStruct((CPAD, 1), jnp.float32),
        ],
    )(hp, tmatT, py_row, pyt_col)


_NB = 256


def _pre_body(lw_ref, ls_ref, e_ref, esum_ref, lse_ref):
    lw = lw_ref[...]
    ls = ls_ref[...]
    mw = jnp.max(lw, axis=0, keepdims=True)
    e = jnp.exp((lw - mw) * (1.0 / TEMPERATURE))
    e_ref[...] = e
    esum_ref[...] = jnp.sum(e, axis=0, keepdims=True)
    ms = jnp.max(ls, axis=0, keepdims=True)
    lse_ref[...] = ms + jnp.log(
        jnp.sum(jnp.exp(ls - ms), axis=0, keepdims=True))


def _pre_tc(lwT, lsT):
    grid = BATCH // _NB
    return pl.pallas_call(
        _pre_body,
        grid=(grid,),
        in_specs=[
            pl.BlockSpec((C, _NB), lambda i: (0, i)),
            pl.BlockSpec((C, _NB), lambda i: (0, i)),
        ],
        out_specs=[
            pl.BlockSpec((C, _NB), lambda i: (0, i)),
            pl.BlockSpec((1, _NB), lambda i: (0, i)),
            pl.BlockSpec((1, _NB), lambda i: (0, i)),
        ],
        out_shape=[
            jax.ShapeDtypeStruct((C, BATCH), jnp.float32),
            jax.ShapeDtypeStruct((1, BATCH), jnp.float32),
            jax.ShapeDtypeStruct((1, BATCH), jnp.float32),
        ],
    )(lwT, lsT)


def _loss_body(e_ref, ls_ref, yt_ref, esum_ref, lse_ref, hi_ref, mid_ref,
               lo_ref, beta_ref, out_ref):
    i = pl.program_id(0)
    e = e_ref[...]
    ls = ls_ref[...]
    ytb = yt_ref[...]
    iota0 = lax.broadcasted_iota(jnp.int32, (C, _NB), 0)
    onehot = jnp.where(iota0 == ytb, 1.0, 0.0).astype(jnp.bfloat16)
    dn = (((0,), (0,)), ((), ()))
    asel = (lax.dot_general(hi_ref[...], onehot, dn,
                            preferred_element_type=jnp.float32)
            + lax.dot_general(mid_ref[...], onehot, dn,
                              preferred_element_type=jnp.float32)
            + lax.dot_general(lo_ref[...], onehot, dn,
                              preferred_element_type=jnp.float32))
    a = asel[:C, :]
    esum = esum_ref[...]
    u = e * a
    s = jnp.sum(u, axis=0, keepdims=True)
    m = jnp.max(u, axis=0, keepdims=True)
    conf = (m / esum) / jnp.maximum(s / esum, 1e-12)
    yh = jnp.min(jnp.where(u == m, iota0, jnp.int32(2**30)), axis=0,
                 keepdims=True)
    sel = iota0 == yh
    bsel = jnp.sum(jnp.where(sel, beta_ref[:C, :], 0.0), axis=0, keepdims=True)
    ssel = jnp.sum(jnp.where(sel, ls, 0.0), axis=0, keepdims=True)
    loss = lse_ref[...] - ssel
    msk = (conf > THRESHOLD * bsel).astype(jnp.float32)
    part = jnp.sum(loss * msk) * (1.0 / BATCH)

    @pl.when(i == 0)
    def _():
        out_ref[0, 0] = 0.0

    out_ref[0, 0] += part


def _loss_tc(e_mat, lsT, yt_row, esum_row, lse_row, am_hi, am_mid, am_lo,
             beta_col):
    grid = BATCH // _NB
    return pl.pallas_call(
        _loss_body,
        grid=(grid,),
        in_specs=[
            pl.BlockSpec((C, _NB), lambda i: (0, i)),
            pl.BlockSpec((C, _NB), lambda i: (0, i)),
            pl.BlockSpec((1, _NB), lambda i: (0, i)),
            pl.BlockSpec((1, _NB), lambda i: (0, i)),
            pl.BlockSpec((1, _NB), lambda i: (0, i)),
            pl.BlockSpec((C, CPAD), lambda i: (0, 0)),
            pl.BlockSpec((C, CPAD), lambda i: (0, 0)),
            pl.BlockSpec((C, CPAD), lambda i: (0, 0)),
            pl.BlockSpec((CPAD, 1), lambda i: (0, 0)),
        ],
        out_specs=pl.BlockSpec((1, 1), lambda i: (0, 0),
                               memory_space=pltpu.MemorySpace.SMEM),
        out_shape=jax.ShapeDtypeStruct((1, 1), jnp.float32),
    )(e_mat, lsT, yt_row, esum_row, lse_row, am_hi, am_mid, am_lo, beta_col)


def kernel(logits_s, logits_w, y_tilde, Y_hat, Y_tilde, T_mat, Py, Py_tilde):
    yh_p = jnp.concatenate(
        [Y_hat.astype(jnp.int32), jnp.zeros((PADN,), jnp.int32)])
    yt_p = jnp.concatenate(
        [Y_tilde.astype(jnp.int32), jnp.full((PADN,), PADROW, jnp.int32)])
    partials = _hist_sc()(yh_p.reshape(NP // SCAT, SCAT),
                          yt_p.reshape(NP // SCAT, SCAT))
    hp = partials.reshape(NC, CPAD, CPAD)
    e_mat, esum_row, lse_row = _pre_tc(logits_w.T, logits_s.T)
    am_hi, am_mid, am_lo, betav = _alpha_tc(hp, T_mat.T, Py.reshape(1, C),
                                            Py_tilde.reshape(C, 1))
    tot = _loss_tc(e_mat, logits_s.T,
                   y_tilde.astype(jnp.int32).reshape(1, BATCH),
                   esum_row, lse_row,
                   am_hi, am_mid, am_lo, betav)
    return tot[0, 0]


# speedup vs baseline: 1.0251x; 1.0251x over previous
import functools

import jax
import jax.numpy as jnp
from jax import lax
from jax.experimental import pallas as pl
from jax.experimental.pallas import tpu as pltpu
from jax.experimental.pallas import tpu_sc as plsc

C = 1000
CP1 = C + 1
CPAD = 1024
NSAMP = 1_000_000
BATCH = 4096
TEMPERATURE = 0.5
THRESHOLD = 0.95

NC, NS = 2, 16
NW = NC * NS
SCAT = 128
NSCAT = 248
CW = NSCAT * SCAT
NP = CW * NW
PADN = NP - NSAMP
PADROW = 1008
TAB = CPAD * CPAD
TPW = TAB // NS
ZCH = 1024


@functools.cache
def _sc_mesh():
    return plsc.VectorSubcoreMesh(core_axis_name="c", subcore_axis_name="s",
                                  num_cores=NC, num_subcores=NS)


def _hist_body(yh_hbm, yt_hbm, out_hbm, yh_v, yt_v, ones_v, zero_v, table,
               zsem, ssem, isem):
    cid = lax.axis_index("c")
    sid = lax.axis_index("s")
    wid = sid * NC + cid

    zeros16 = jnp.zeros((16,), jnp.float32)
    ones16 = jnp.ones((16,), jnp.float32)

    @pl.loop(0, ZCH // 16)
    def _fill_zero(i):
        zero_v[pl.ds(i * 16, 16)] = zeros16

    for k in range(SCAT // 16):
        ones_v[pl.ds(k * 16, 16)] = ones16

    QS = (64, 64, 64, 56)
    QO = (0, 64, 128, 192)

    def _load_q(q):
        base = wid * NSCAT + QO[q]
        return (pltpu.async_copy(yh_hbm.at[pl.ds(base, QS[q])],
                                 yh_v.at[pl.ds(QO[q], QS[q])], isem),
                pltpu.async_copy(yt_hbm.at[pl.ds(base, QS[q])],
                                 yt_v.at[pl.ds(QO[q], QS[q])], isem))

    inflight = _load_q(0)

    zcopies = [
        pltpu.async_copy(zero_v, table.at[pl.ds(sid * TPW + z * ZCH, ZCH)],
                         zsem)
        for z in range(TPW // ZCH)
    ]
    for zc in zcopies:
        zc.wait()
    plsc.subcore_barrier()

    GRP = 8
    for q in range(4):
        for ic in inflight:
            ic.wait()
        if q + 1 < 4:
            inflight = _load_q(q + 1)

        @pl.loop(0, QS[q] // GRP)
        def _scatter(g):
            copies = []
            for j2 in range(GRP):
                j = QO[q] + g * GRP + j2
                for k in range(SCAT // 16):
                    sl = pl.ds(k * 16, 16)
                    yh_v[j, sl] = yt_v[j, sl] * CPAD + yh_v[j, sl]
                copies.append(
                    pltpu.async_copy(ones_v, table.at[yh_v.at[j]], ssem,
                                     add=True))
            for cp in copies:
                cp.wait()

    plsc.subcore_barrier()
    pltpu.sync_copy(table.at[pl.ds(sid * TPW, TPW)], out_hbm.at[cid, sid])


@functools.cache
def _hist_sc():
    return pl.kernel(
        _hist_body,
        out_type=jax.ShapeDtypeStruct((NC, NS, TPW), jnp.float32),
        mesh=_sc_mesh(),
        scratch_types=[
            pltpu.VMEM((NSCAT, SCAT), jnp.int32),
            pltpu.VMEM((NSCAT, SCAT), jnp.int32),
            pltpu.VMEM((SCAT,), jnp.float32),
            pltpu.VMEM((ZCH,), jnp.float32),
            pltpu.VMEM_SHARED((TAB,), jnp.float32),
            pltpu.SemaphoreType.DMA,
            pltpu.SemaphoreType.DMA,
            pltpu.SemaphoreType.DMA,
        ],
    )


def _alpha_body(hp_ref, tmat_ref, py_ref, pyt_ref, hi_ref, mid_ref, lo_ref,
                beta_ref):
    h = hp_ref[0] + hp_ref[1]
    cnt = jnp.sum(h[:C, :], axis=0, keepdims=True)
    bmax = jnp.max(cnt)
    beta_ref[...] = (cnt / (2.0 * bmax - cnt)).T
    hT = h[:C, :C]
    un = h[:C, C:CP1]
    py = py_ref[...]
    tden = hT + un * py + 1.0
    tden = tden / jnp.sum(tden, axis=0, keepdims=True)
    tT = tmat_ref[...]
    alphaT = tT / tden
    err = (1.0 - THRESHOLD) * pyt_ref[...] / py
    am = jnp.where(jnp.abs(tT - tden) > err, alphaT, 1.0)
    amp = jnp.concatenate([am, jnp.zeros((C, CPAD - C), jnp.float32)], axis=1)
    hi = amp.astype(jnp.bfloat16)
    r1 = amp - hi.astype(jnp.float32)
    mid = r1.astype(jnp.bfloat16)
    lo = (r1 - mid.astype(jnp.float32)).astype(jnp.bfloat16)
    hi_ref[...] = hi
    mid_ref[...] = mid
    lo_ref[...] = lo


def _alpha_tc(hp, tmatT, py_row, pyt_col):
    return pl.pallas_call(
        _alpha_body,
        out_shape=[
            jax.ShapeDtypeStruct((C, CPAD), jnp.bfloat16),
            jax.ShapeDtypeStruct((C, CPAD), jnp.bfloat16),
            jax.ShapeDtypeStruct((C, CPAD), jnp.bfloat16),
            jax.ShapeDtypeStruct((CPAD, 1), jnp.float32),
        ],
    )(hp, tmatT, py_row, pyt_col)


_NB = 512


def _pre_body(lw_ref, ls_ref, e_ref, esum_ref, lse_ref):
    lw = lw_ref[...]
    ls = ls_ref[...]
    mw = jnp.max(lw, axis=0, keepdims=True)
    e = jnp.exp((lw - mw) * (1.0 / TEMPERATURE))
    e_ref[...] = e
    esum_ref[...] = jnp.sum(e, axis=0, keepdims=True)
    ms = jnp.max(ls, axis=0, keepdims=True)
    lse_ref[...] = ms + jnp.log(
        jnp.sum(jnp.exp(ls - ms), axis=0, keepdims=True))


def _pre_tc(lwT, lsT):
    grid = BATCH // _NB
    return pl.pallas_call(
        _pre_body,
        grid=(grid,),
        in_specs=[
            pl.BlockSpec((C, _NB), lambda i: (0, i)),
            pl.BlockSpec((C, _NB), lambda i: (0, i)),
        ],
        out_specs=[
            pl.BlockSpec((C, _NB), lambda i: (0, i)),
            pl.BlockSpec((1, _NB), lambda i: (0, i)),
            pl.BlockSpec((1, _NB), lambda i: (0, i)),
        ],
        out_shape=[
            jax.ShapeDtypeStruct((C, BATCH), jnp.float32),
            jax.ShapeDtypeStruct((1, BATCH), jnp.float32),
            jax.ShapeDtypeStruct((1, BATCH), jnp.float32),
        ],
    )(lwT, lsT)


def _loss_body(e_ref, ls_ref, yt_ref, esum_ref, lse_ref, hi_ref, mid_ref,
               lo_ref, beta_ref, out_ref):
    i = pl.program_id(0)
    e = e_ref[...]
    ls = ls_ref[...]
    ytb = yt_ref[...]
    iota0 = lax.broadcasted_iota(jnp.int32, (C, _NB), 0)
    onehot = jnp.where(iota0 == ytb, 1.0, 0.0).astype(jnp.bfloat16)
    dn = (((0,), (0,)), ((), ()))
    asel = (lax.dot_general(hi_ref[...], onehot, dn,
                            preferred_element_type=jnp.float32)
            + lax.dot_general(mid_ref[...], onehot, dn,
                              preferred_element_type=jnp.float32)
            + lax.dot_general(lo_ref[...], onehot, dn,
                              preferred_element_type=jnp.float32))
    a = asel[:C, :]
    esum = esum_ref[...]
    u = e * a
    s = jnp.sum(u, axis=0, keepdims=True)
    m = jnp.max(u, axis=0, keepdims=True)
    conf = (m / esum) / jnp.maximum(s / esum, 1e-12)
    yh = jnp.min(jnp.where(u == m, iota0, jnp.int32(2**30)), axis=0,
                 keepdims=True)
    sel = iota0 == yh
    bsel = jnp.sum(jnp.where(sel, beta_ref[:C, :], 0.0), axis=0, keepdims=True)
    ssel = jnp.sum(jnp.where(sel, ls, 0.0), axis=0, keepdims=True)
    loss = lse_ref[...] - ssel
    msk = (conf > THRESHOLD * bsel).astype(jnp.float32)
    part = jnp.sum(loss * msk) * (1.0 / BATCH)

    @pl.when(i == 0)
    def _():
        out_ref[0, 0] = 0.0

    out_ref[0, 0] += part


def _loss_tc(e_mat, lsT, yt_row, esum_row, lse_row, am_hi, am_mid, am_lo,
             beta_col):
    grid = BATCH // _NB
    return pl.pallas_call(
        _loss_body,
        grid=(grid,),
        in_specs=[
            pl.BlockSpec((C, _NB), lambda i: (0, i)),
            pl.BlockSpec((C, _NB), lambda i: (0, i)),
            pl.BlockSpec((1, _NB), lambda i: (0, i)),
            pl.BlockSpec((1, _NB), lambda i: (0, i)),
            pl.BlockSpec((1, _NB), lambda i: (0, i)),
            pl.BlockSpec((C, CPAD), lambda i: (0, 0)),
            pl.BlockSpec((C, CPAD), lambda i: (0, 0)),
            pl.BlockSpec((C, CPAD), lambda i: (0, 0)),
            pl.BlockSpec((CPAD, 1), lambda i: (0, 0)),
        ],
        out_specs=pl.BlockSpec((1, 1), lambda i: (0, 0),
                               memory_space=pltpu.MemorySpace.SMEM),
        out_shape=jax.ShapeDtypeStruct((1, 1), jnp.float32),
    )(e_mat, lsT, yt_row, esum_row, lse_row, am_hi, am_mid, am_lo, beta_col)


def kernel(logits_s, logits_w, y_tilde, Y_hat, Y_tilde, T_mat, Py, Py_tilde):
    yh_p = jnp.concatenate(
        [Y_hat.astype(jnp.int32), jnp.zeros((PADN,), jnp.int32)])
    yt_p = jnp.concatenate(
        [Y_tilde.astype(jnp.int32), jnp.full((PADN,), PADROW, jnp.int32)])
    partials = _hist_sc()(yh_p.reshape(NP // SCAT, SCAT),
                          yt_p.reshape(NP // SCAT, SCAT))
    hp = partials.reshape(NC, CPAD, CPAD)
    e_mat, esum_row, lse_row = _pre_tc(logits_w.T, logits_s.T)
    am_hi, am_mid, am_lo, betav = _alpha_tc(hp, T_mat.T, Py.reshape(1, C),
                                            Py_tilde.reshape(C, 1))
    tot = _loss_tc(e_mat, logits_s.T,
                   y_tilde.astype(jnp.int32).reshape(1, BATCH),
                   esum_row, lse_row,
                   am_hi, am_mid, am_lo, betav)
    return tot[0, 0]


# speedup vs baseline: 1.0287x; 1.0035x over previous
import functools

import jax
import jax.numpy as jnp
from jax import lax
from jax.experimental import pallas as pl
from jax.experimental.pallas import tpu as pltpu
from jax.experimental.pallas import tpu_sc as plsc

C = 1000
CP1 = C + 1
CPAD = 1024
NSAMP = 1_000_000
BATCH = 4096
TEMPERATURE = 0.5
THRESHOLD = 0.95

NC, NS = 2, 16
NW = NC * NS
SCAT = 128
NSCAT = 248
CW = NSCAT * SCAT
NP = CW * NW
PADN = NP - NSAMP
PADROW = 1008
TAB = CPAD * CPAD
TPW = TAB // NS
ZCH = 1024


@functools.cache
def _sc_mesh():
    return plsc.VectorSubcoreMesh(core_axis_name="c", subcore_axis_name="s",
                                  num_cores=NC, num_subcores=NS)


def _hist_body(yh_hbm, yt_hbm, out_hbm, yh_v, yt_v, ones_v, zero_v, table,
               zsem, ssem, isem):
    cid = lax.axis_index("c")
    sid = lax.axis_index("s")
    wid = sid * NC + cid

    zeros16 = jnp.zeros((16,), jnp.float32)
    ones16 = jnp.ones((16,), jnp.float32)

    @pl.loop(0, ZCH // 16)
    def _fill_zero(i):
        zero_v[pl.ds(i * 16, 16)] = zeros16

    for k in range(SCAT // 16):
        ones_v[pl.ds(k * 16, 16)] = ones16

    QS = (64, 64, 64, 56)
    QO = (0, 64, 128, 192)

    def _load_q(q):
        base = wid * NSCAT + QO[q]
        return (pltpu.async_copy(yh_hbm.at[pl.ds(base, QS[q])],
                                 yh_v.at[pl.ds(QO[q], QS[q])], isem),
                pltpu.async_copy(yt_hbm.at[pl.ds(base, QS[q])],
                                 yt_v.at[pl.ds(QO[q], QS[q])], isem))

    inflight = _load_q(0)

    zcopies = [
        pltpu.async_copy(zero_v, table.at[pl.ds(sid * TPW + z * ZCH, ZCH)],
                         zsem)
        for z in range(TPW // ZCH)
    ]
    for zc in zcopies:
        zc.wait()
    plsc.subcore_barrier()

    GRP = 8
    for q in range(4):
        for ic in inflight:
            ic.wait()
        if q + 1 < 4:
            inflight = _load_q(q + 1)

        @pl.loop(0, QS[q] // GRP)
        def _scatter(g):
            copies = []
            for j2 in range(GRP):
                j = QO[q] + g * GRP + j2
                for k in range(SCAT // 16):
                    sl = pl.ds(k * 16, 16)
                    yh_v[j, sl] = yt_v[j, sl] * CPAD + yh_v[j, sl]
                copies.append(
                    pltpu.async_copy(ones_v, table.at[yh_v.at[j]], ssem,
                                     add=True))
            for cp in copies:
                cp.wait()

    plsc.subcore_barrier()
    pltpu.sync_copy(table.at[pl.ds(sid * TPW, TPW)], out_hbm.at[cid, sid])


@functools.cache
def _hist_sc():
    return pl.kernel(
        _hist_body,
        out_type=jax.ShapeDtypeStruct((NC, NS, TPW), jnp.float32),
        mesh=_sc_mesh(),
        scratch_types=[
            pltpu.VMEM((NSCAT, SCAT), jnp.int32),
            pltpu.VMEM((NSCAT, SCAT), jnp.int32),
            pltpu.VMEM((SCAT,), jnp.float32),
            pltpu.VMEM((ZCH,), jnp.float32),
            pltpu.VMEM_SHARED((TAB,), jnp.float32),
            pltpu.SemaphoreType.DMA,
            pltpu.SemaphoreType.DMA,
            pltpu.SemaphoreType.DMA,
        ],
    )


def _alpha_body(hp_ref, tmat_ref, py_ref, pyt_ref, hi_ref, mid_ref, lo_ref,
                beta_ref):
    h = hp_ref[0] + hp_ref[1]
    cnt = jnp.sum(h[:C, :], axis=0, keepdims=True)
    bmax = jnp.max(cnt)
    beta_ref[...] = (cnt / (2.0 * bmax - cnt)).T
    hT = h[:C, :C]
    un = h[:C, C:CP1]
    py = py_ref[...]
    tden = hT + un * py + 1.0
    tden = tden / jnp.sum(tden, axis=0, keepdims=True)
    tT = tmat_ref[...]
    alphaT = tT / tden
    err = (1.0 - THRESHOLD) * pyt_ref[...] / py
    am = jnp.where(jnp.abs(tT - tden) > err, alphaT, 1.0)
    amp = jnp.concatenate([am, jnp.zeros((C, CPAD - C), jnp.float32)], axis=1)
    hi = amp.astype(jnp.bfloat16)
    r1 = amp - hi.astype(jnp.float32)
    mid = r1.astype(jnp.bfloat16)
    lo = (r1 - mid.astype(jnp.float32)).astype(jnp.bfloat16)
    hi_ref[...] = hi
    mid_ref[...] = mid
    lo_ref[...] = lo


def _alpha_tc(hp, tmatT, py_row, pyt_col):
    return pl.pallas_call(
        _alpha_body,
        out_shape=[
            jax.ShapeDtypeStruct((C, CPAD), jnp.bfloat16),
            jax.ShapeDtypeStruct((C, CPAD), jnp.bfloat16),
            jax.ShapeDtypeStruct((C, CPAD), jnp.bfloat16),
            jax.ShapeDtypeStruct((CPAD, 1), jnp.float32),
        ],
    )(hp, tmatT, py_row, pyt_col)


_NB = 1024


def _pre_body(lw_ref, ls_ref, e_ref, esum_ref, lse_ref):
    lw = lw_ref[...]
    ls = ls_ref[...]
    mw = jnp.max(lw, axis=0, keepdims=True)
    e = jnp.exp((lw - mw) * (1.0 / TEMPERATURE))
    e_ref[...] = e
    esum_ref[...] = jnp.sum(e, axis=0, keepdims=True)
    ms = jnp.max(ls, axis=0, keepdims=True)
    lse_ref[...] = ms + jnp.log(
        jnp.sum(jnp.exp(ls - ms), axis=0, keepdims=True))


def _pre_tc(lwT, lsT):
    grid = BATCH // _NB
    return pl.pallas_call(
        _pre_body,
        grid=(grid,),
        in_specs=[
            pl.BlockSpec((C, _NB), lambda i: (0, i)),
            pl.BlockSpec((C, _NB), lambda i: (0, i)),
        ],
        out_specs=[
            pl.BlockSpec((C, _NB), lambda i: (0, i)),
            pl.BlockSpec((1, _NB), lambda i: (0, i)),
            pl.BlockSpec((1, _NB), lambda i: (0, i)),
        ],
        out_shape=[
            jax.ShapeDtypeStruct((C, BATCH), jnp.float32),
            jax.ShapeDtypeStruct((1, BATCH), jnp.float32),
            jax.ShapeDtypeStruct((1, BATCH), jnp.float32),
        ],
    )(lwT, lsT)


def _loss_body(e_ref, ls_ref, yt_ref, esum_ref, lse_ref, hi_ref, mid_ref,
               lo_ref, beta_ref, out_ref):
    i = pl.program_id(0)
    e = e_ref[...]
    ls = ls_ref[...]
    ytb = yt_ref[...]
    iota0 = lax.broadcasted_iota(jnp.int32, (C, _NB), 0)
    onehot = jnp.where(iota0 == ytb, 1.0, 0.0).astype(jnp.bfloat16)
    dn = (((0,), (0,)), ((), ()))
    asel = (lax.dot_general(hi_ref[...], onehot, dn,
                            preferred_element_type=jnp.float32)
            + lax.dot_general(mid_ref[...], onehot, dn,
                              preferred_element_type=jnp.float32)
            + lax.dot_general(lo_ref[...], onehot, dn,
                              preferred_element_type=jnp.float32))
    a = asel[:C, :]
    esum = esum_ref[...]
    u = e * a
    s = jnp.sum(u, axis=0, keepdims=True)
    m = jnp.max(u, axis=0, keepdims=True)
    conf = (m / esum) / jnp.maximum(s / esum, 1e-12)
    yh = jnp.min(jnp.where(u == m, iota0, jnp.int32(2**30)), axis=0,
                 keepdims=True)
    sel = iota0 == yh
    bsel = jnp.sum(jnp.where(sel, beta_ref[:C, :], 0.0), axis=0, keepdims=True)
    ssel = jnp.sum(jnp.where(sel, ls, 0.0), axis=0, keepdims=True)
    loss = lse_ref[...] - ssel
    msk = (conf > THRESHOLD * bsel).astype(jnp.float32)
    part = jnp.sum(loss * msk) * (1.0 / BATCH)

    @pl.when(i == 0)
    def _():
        out_ref[0, 0] = 0.0

    out_ref[0, 0] += part


def _loss_tc(e_mat, lsT, yt_row, esum_row, lse_row, am_hi, am_mid, am_lo,
             beta_col):
    grid = BATCH // _NB
    return pl.pallas_call(
        _loss_body,
        grid=(grid,),
        in_specs=[
            pl.BlockSpec((C, _NB), lambda i: (0, i)),
            pl.BlockSpec((C, _NB), lambda i: (0, i)),
            pl.BlockSpec((1, _NB), lambda i: (0, i)),
            pl.BlockSpec((1, _NB), lambda i: (0, i)),
            pl.BlockSpec((1, _NB), lambda i: (0, i)),
            pl.BlockSpec((C, CPAD), lambda i: (0, 0)),
            pl.BlockSpec((C, CPAD), lambda i: (0, 0)),
            pl.BlockSpec((C, CPAD), lambda i: (0, 0)),
            pl.BlockSpec((CPAD, 1), lambda i: (0, 0)),
        ],
        out_specs=pl.BlockSpec((1, 1), lambda i: (0, 0),
                               memory_space=pltpu.MemorySpace.SMEM),
        out_shape=jax.ShapeDtypeStruct((1, 1), jnp.float32),
    )(e_mat, lsT, yt_row, esum_row, lse_row, am_hi, am_mid, am_lo, beta_col)


def kernel(logits_s, logits_w, y_tilde, Y_hat, Y_tilde, T_mat, Py, Py_tilde):
    yh_p = jnp.concatenate(
        [Y_hat.astype(jnp.int32), jnp.zeros((PADN,), jnp.int32)])
    yt_p = jnp.concatenate(
        [Y_tilde.astype(jnp.int32), jnp.full((PADN,), PADROW, jnp.int32)])
    partials = _hist_sc()(yh_p.reshape(NP // SCAT, SCAT),
                          yt_p.reshape(NP // SCAT, SCAT))
    hp = partials.reshape(NC, CPAD, CPAD)
    e_mat, esum_row, lse_row = _pre_tc(logits_w.T, logits_s.T)
    am_hi, am_mid, am_lo, betav = _alpha_tc(hp, T_mat.T, Py.reshape(1, C),
                                            Py_tilde.reshape(C, 1))
    tot = _loss_tc(e_mat, logits_s.T,
                   y_tilde.astype(jnp.int32).reshape(1, BATCH),
                   esum_row, lse_row,
                   am_hi, am_mid, am_lo, betav)
    return tot[0, 0]
